# Initial kernel scaffold; baseline (speedup 1.0000x reference)
import jax
import jax.numpy as jnp
from jax.experimental import pallas as pl


def kernel(x, Wfc0, al0, ar0, Wres0, b0, Wfc1, al1, ar1, Wres1, b1, Wfc2, al2, ar2, Wres2, b2):
    raise NotImplementedError("write your pallas kernel here")


# speedup vs baseline: 1.0902x; 1.0902x over previous
import functools

import jax
import jax.numpy as jnp
from jax.experimental import pallas as pl
from jax.experimental.pallas import tpu as pltpu

N = 512
D = 512
HEADS = 5


def _gat_stack_kernel(h_in, wfc, al, ar, wres, b, out, h_s, acc):
    i = pl.program_id(0)

    @pl.when(i == 0)
    def _():
        h_s[...] = h_in[...]

    h = h_s[...]
    is_hidden = i < 2

    acc[...] = jnp.zeros((N, D), jnp.float32)
    for hd in range(HEADS):
        wfc_h = wfc[0, hd * D:(hd + 1) * D, :]
        feat = jax.lax.dot_general(
            h, wfc_h, (((1,), (1,)), ((), ())),
            preferred_element_type=jnp.float32)

        al_row = al[0, hd:hd + 1, :]
        ar_row = ar[0, hd:hd + 1, :]
        el = jax.lax.dot_general(
            feat, al_row, (((1,), (1,)), ((), ())),
            preferred_element_type=jnp.float32)
        er = jax.lax.dot_general(
            ar_row, feat, (((1,), (1,)), ((), ())),
            preferred_element_type=jnp.float32)

        e = el + er
        e = jnp.where(e > 0, e, 0.2 * e)
        m = jnp.max(e, axis=0, keepdims=True)
        p = jnp.exp(e - m)
        denom = jnp.sum(p, axis=0, keepdims=True)
        p = p * (1.0 / denom)

        rst = jax.lax.dot_general(
            p, feat, (((0,), (0,)), ((), ())),
            preferred_element_type=jnp.float32)

        wres_h = wres[0, hd * D:(hd + 1) * D, :]
        res = jax.lax.dot_general(
            h, wres_h, (((1,), (1,)), ((), ())),
            preferred_element_type=jnp.float32)

        t = rst + res + b[0, hd:hd + 1, :]
        t = jnp.where(is_hidden, jnp.maximum(t, 0.0), t)
        acc[...] += t

    h_new = acc[...] * (1.0 / HEADS)

    @pl.when(is_hidden)
    def _():
        h_s[...] = h_new

    @pl.when(jnp.logical_not(is_hidden))
    def _():
        out[...] = h_new


@functools.partial(jax.jit, static_argnames=("interpret",))
def kernel(x, Wfc0, al0, ar0, Wres0, b0, Wfc1, al1, ar1, Wres1, b1,
           Wfc2, al2, ar2, Wres2, b2, interpret=False):
    B, C, Hs, Ws = x.shape
    h0 = x.reshape(C, Hs * Ws).T

    wfc = jnp.stack([Wfc0, Wfc1, Wfc2])
    wres = jnp.stack([Wres0, Wres1, Wres2])
    al = jnp.stack([al0, al1, al2])
    ar = jnp.stack([ar0, ar1, ar2])
    b = jnp.stack([b0, b1, b2]).reshape(3, HEADS, D)

    hidden = pl.pallas_call(
        _gat_stack_kernel,
        grid=(3,),
        in_specs=[
            pl.BlockSpec((N, D), lambda i: (0, 0)),
            pl.BlockSpec((1, HEADS * D, D), lambda i: (i, 0, 0)),
            pl.BlockSpec((1, HEADS, D), lambda i: (i, 0, 0)),
            pl.BlockSpec((1, HEADS, D), lambda i: (i, 0, 0)),
            pl.BlockSpec((1, HEADS * D, D), lambda i: (i, 0, 0)),
            pl.BlockSpec((1, HEADS, D), lambda i: (i, 0, 0)),
        ],
        out_specs=pl.BlockSpec((N, D), lambda i: (0, 0)),
        out_shape=jax.ShapeDtypeStruct((N, D), jnp.float32),
        scratch_shapes=[
            pltpu.VMEM((N, D), jnp.float32),
            pltpu.VMEM((N, D), jnp.float32),
        ],
        interpret=interpret,
    )(h0, wfc, al, ar, wres, b)

    return hidden.T.reshape(B, C, Hs, Ws)


# speedup vs baseline: 1.8834x; 1.7275x over previous
import functools

import jax
import jax.numpy as jnp
from jax.experimental import pallas as pl
from jax.experimental.pallas import tpu as pltpu

N = 512
D = 512
HEADS = 5


def _gat_stack_kernel(h_in, wfc0, al0, ar0, wres0, b0,
                      wfc1, al1, ar1, wres1, b1,
                      wfc2, al2, ar2, wres2, b2, out):
    h = h_in[...]
    layers = ((wfc0, al0, ar0, wres0, b0, True),
              (wfc1, al1, ar1, wres1, b1, True),
              (wfc2, al2, ar2, wres2, b2, False))
    for wfc, al, ar, wres, b, act in layers:
        featall = jax.lax.dot_general(
            h, wfc[...], (((1,), (1,)), ((), ())),
            preferred_element_type=jnp.float32)
        resall = jax.lax.dot_general(
            h, wres[...], (((1,), (1,)), ((), ())),
            preferred_element_type=jnp.float32)
        acc = None
        for hd in range(HEADS):
            feat = featall[:, hd * D:(hd + 1) * D]
            al_row = al[hd:hd + 1, :]
            ar_row = ar[hd:hd + 1, :]
            el_col = jnp.sum(feat * al_row, axis=1, keepdims=True)
            er_col = jnp.sum(feat * ar_row, axis=1, keepdims=True)
            el_row = jax.lax.transpose(el_col, (1, 0))

            e2 = er_col + el_row
            e2 = jnp.where(e2 > 0, e2, 0.2 * e2)
            m = jnp.max(e2, axis=1, keepdims=True)
            p2 = jnp.exp(e2 - m)
            denom = jnp.sum(p2, axis=1, keepdims=True)
            p2 = p2 * (1.0 / denom)

            rst = jax.lax.dot_general(
                p2, feat, (((1,), (0,)), ((), ())),
                preferred_element_type=jnp.float32)

            t = rst + resall[:, hd * D:(hd + 1) * D] + b[hd:hd + 1, :]
            if act:
                t = jnp.maximum(t, 0.0)
            acc = t if acc is None else acc + t
        h = acc * (1.0 / HEADS)
    out[...] = h


@functools.partial(jax.jit, static_argnames=("interpret",))
def kernel(x, Wfc0, al0, ar0, Wres0, b0, Wfc1, al1, ar1, Wres1, b1,
           Wfc2, al2, ar2, Wres2, b2, interpret=False):
    B, C, Hs, Ws = x.shape
    h0 = x.reshape(C, Hs * Ws).T

    hidden = pl.pallas_call(
        _gat_stack_kernel,
        out_shape=jax.ShapeDtypeStruct((N, D), jnp.float32),
        interpret=interpret,
    )(h0,
      Wfc0, al0, ar0, Wres0, b0.reshape(HEADS, D),
      Wfc1, al1, ar1, Wres1, b1.reshape(HEADS, D),
      Wfc2, al2, ar2, Wres2, b2.reshape(HEADS, D))

    return hidden.T.reshape(B, C, Hs, Ws)
